# baseline (speedup 1.0000x reference)
import functools

import jax
import jax.numpy as jnp
from jax import lax
from jax.experimental import pallas as pl
from jax.experimental.pallas import tpu as pltpu
from jax.experimental.pallas import tpu_sc as plsc

_NC = 2
_NS = 16
_LANES = 16


def _make_pool(B, L, E):
    NW = _NC * _NS
    BW = B // NW
    HL = L // 2
    CH = 64
    NSTEP = BW // CH
    EG = E // _LANES
    NBUF = 8
    GRP = NBUF // 2
    PAIRS = CH // GRP
    UNROLL = 5

    mesh = plsc.VectorSubcoreMesh(
        core_axis_name="c", subcore_axis_name="s",
        num_cores=_NC, num_subcores=_NS)

    @functools.partial(
        pl.kernel,
        mesh=mesh,
        compiler_params=pltpu.CompilerParams(
            use_tc_tiling_on_sc=False, needs_layout_passes=False),
        out_type=jax.ShapeDtypeStruct((B, E), jnp.float32),
        scratch_types=[
            pltpu.VMEM((2 * CH, HL), jnp.int32),
        ] + [pltpu.VMEM((HL, E), jnp.bfloat16)] * NBUF
          + [pltpu.VMEM((CH, E), jnp.float32)]
          + [pltpu.SemaphoreType.DMA] * NBUF,
    )
    def pool(ids_hbm, tab_hbm, y_hbm, idx_v, *rest):
        bufs = rest[:NBUF]
        out_v = rest[NBUF]
        sems = rest[NBUF + 1:]
        wid = lax.axis_index("s") * _NC + lax.axis_index("c")

        def accumulate(rows, accs):
            def row_body(j, accs):
                a = list(accs)
                for u in range(UNROLL):
                    r = j * UNROLL + u
                    for c in range(EG // 2):
                        w = plsc.bitcast(
                            rows[r, pl.ds(2 * _LANES * c, 2 * _LANES)],
                            jnp.int32)
                        lo = plsc.bitcast(w << 16, jnp.float32)
                        hi = plsc.bitcast(w & jnp.int32(-65536), jnp.float32)
                        a[2 * c] = a[2 * c] + lo
                        a[2 * c + 1] = a[2 * c + 1] + hi
                return tuple(a)
            return lax.fori_loop(0, HL // UNROLL, row_body, accs)

        for step in range(NSTEP):
            b0 = wid * BW + step * CH
            pltpu.sync_copy(ids_hbm.at[pl.ds(b0 * 2, 2 * CH)], idx_v)
            for q in range(NBUF):
                pltpu.async_copy(tab_hbm.at[idx_v.at[q]], bufs[q], sems[q])

            def pair_body(p, carry):
                for pair in range(GRP):
                    accs = tuple(jnp.zeros((_LANES,), jnp.float32)
                                 for _ in range(EG))
                    for half in range(2):
                        q = 2 * pair + half
                        h = NBUF * p + q
                        pltpu.make_async_copy(
                            tab_hbm.at[idx_v.at[h]], bufs[q], sems[q]
                        ).wait()
                        accs = accumulate(bufs[q], accs)

                        @pl.when(p < PAIRS - 1)
                        def _():
                            pltpu.async_copy(
                                tab_hbm.at[idx_v.at[h + NBUF]],
                                bufs[q], sems[q])
                    for c in range(EG):
                        out_v[GRP * p + pair,
                              pl.ds(_LANES * c, _LANES)] = accs[c]
                return carry

            lax.fori_loop(0, PAIRS, pair_body, 0)
            pltpu.sync_copy(out_v, y_hbm.at[pl.ds(b0, CH)])

    return pool


def _tr_body(x_ref, o_ref):
    o_ref[:, 0:64] = x_ref[...].astype(jnp.bfloat16).T


def _transpose_pack(embT):
    E, V = embT.shape
    TB = 4096
    grid = (V + TB - 1) // TB
    return pl.pallas_call(
        _tr_body,
        grid=(grid,),
        in_specs=[pl.BlockSpec((E, TB), lambda i: (0, i))],
        out_specs=pl.BlockSpec((TB, 2 * E), lambda i: (i, 0)),
        out_shape=jax.ShapeDtypeStruct((V, 2 * E), jnp.bfloat16),
    )(embT)


def _mm_body(y_ref, wt_ref, b_ref, o_ref):
    o_ref[...] = jnp.dot(
        y_ref[...], wt_ref[...],
        preferred_element_type=jnp.float32,
        precision=lax.Precision.HIGHEST,
    ) + b_ref[...]


def _matmul(y, Wt, b2):
    B, E = y.shape
    N = Wt.shape[1]
    BB = 1024
    return pl.pallas_call(
        _mm_body,
        grid=(B // BB,),
        in_specs=[
            pl.BlockSpec((BB, E), lambda i: (i, 0)),
            pl.BlockSpec((E, N), lambda i: (0, 0)),
            pl.BlockSpec((1, N), lambda i: (0, 0)),
        ],
        out_specs=pl.BlockSpec((BB, N), lambda i: (i, 0)),
        out_shape=jax.ShapeDtypeStruct((B, N), jnp.float32),
    )(y, Wt, b2)


def kernel(input, embed, W, b):
    B, L = input.shape
    V, E = embed.shape
    ids2 = (input.astype(jnp.int32) * 2).reshape(2 * B, L // 2)
    packed = _transpose_pack(embed.T)
    table2 = packed.reshape(2 * packed.shape[0], E)
    y = _make_pool(B, L, E)(ids2, table2)
    perm = jnp.concatenate([
        jnp.arange(0, 32, 2), jnp.arange(1, 32, 2),
        jnp.arange(32, 64, 2), jnp.arange(33, 64, 2)])
    return _matmul(y, W.T[perm, :], b.reshape(1, -1))


# speedup vs baseline: 2.2364x; 2.2364x over previous
import functools

import jax
import jax.numpy as jnp
from jax import lax
from jax.experimental import pallas as pl
from jax.experimental.pallas import tpu as pltpu
from jax.experimental.pallas import tpu_sc as plsc

_NC = 2
_NS = 16
_LANES = 16


def _make_pool(B, L, E):
    NW = _NC * _NS
    BW = B // NW
    HL = L // 2
    CH = 64
    NSTEP = BW // CH
    EG = E // _LANES
    NBUF = 8
    GRP = NBUF // 2
    PAIRS = CH // GRP
    UNROLL = 5

    mesh = plsc.VectorSubcoreMesh(
        core_axis_name="c", subcore_axis_name="s",
        num_cores=_NC, num_subcores=_NS)

    @functools.partial(
        pl.kernel,
        mesh=mesh,
        compiler_params=pltpu.CompilerParams(use_tc_tiling_on_sc=False),
        out_type=jax.ShapeDtypeStruct((B, E), jnp.float32),
        scratch_types=[
            pltpu.VMEM((2 * CH, HL), jnp.int32),
        ] + [pltpu.VMEM((HL, E // 2), jnp.int32)] * NBUF
          + [pltpu.VMEM((CH, E), jnp.float32)]
          + [pltpu.SemaphoreType.DMA] * NBUF,
    )
    def pool(ids_hbm, tab_hbm, y_hbm, idx_v, *rest):
        bufs = rest[:NBUF]
        out_v = rest[NBUF]
        sems = rest[NBUF + 1:]
        wid = lax.axis_index("s") * _NC + lax.axis_index("c")

        def accumulate(rows, accs):
            def row_body(j, accs):
                a = list(accs)
                for u in range(UNROLL):
                    r = j * UNROLL + u
                    for c in range(EG // 2):
                        w = rows[r, pl.ds(_LANES * c, _LANES)]
                        lo = lax.bitcast_convert_type(
                            lax.shift_left(w, 16), jnp.float32)
                        hi = lax.bitcast_convert_type(
                            w & jnp.int32(-65536), jnp.float32)
                        a[2 * c] = a[2 * c] + lo
                        a[2 * c + 1] = a[2 * c + 1] + hi
                return tuple(a)
            return lax.fori_loop(0, HL // UNROLL, row_body, accs)

        for step in range(NSTEP):
            b0 = wid * BW + step * CH
            pltpu.sync_copy(ids_hbm.at[pl.ds(b0 * 2, 2 * CH)], idx_v)
            for q in range(NBUF):
                pltpu.async_copy(tab_hbm.at[idx_v.at[q]], bufs[q], sems[q])

            def pair_body(p, carry):
                for pair in range(GRP):
                    accs = tuple(jnp.zeros((_LANES,), jnp.float32)
                                 for _ in range(EG))
                    for half in range(2):
                        q = 2 * pair + half
                        h = NBUF * p + q
                        pltpu.make_async_copy(
                            tab_hbm.at[idx_v.at[h]], bufs[q], sems[q]
                        ).wait()
                        accs = accumulate(bufs[q], accs)

                        @pl.when(p < PAIRS - 1)
                        def _():
                            pltpu.async_copy(
                                tab_hbm.at[idx_v.at[h + NBUF]],
                                bufs[q], sems[q])
                    for c in range(EG):
                        out_v[GRP * p + pair,
                              pl.ds(_LANES * c, _LANES)] = accs[c]
                return carry

            lax.fori_loop(0, PAIRS, pair_body, 0)
            pltpu.sync_copy(out_v, y_hbm.at[pl.ds(b0, CH)])

    return pool


_TTOK = 4096
_QT = _TTOK // 4


def _tr_body(x_ref, o_ref):
    xi = lax.bitcast_convert_type(x_ref[...], jnp.int32)
    x3 = xi.reshape(32, 2, _TTOK)
    ev = x3[:, 0, :]
    od = x3[:, 1, :]
    w = lax.shift_right_logical(ev, 16) | (od & jnp.int32(-65536))
    for q in range(4):
        o_ref[:, 32 * q:32 * (q + 1)] = w[:, q * _QT:(q + 1) * _QT].T


def _transpose_pack(embT):
    E, V = embT.shape
    grid = (V + _TTOK - 1) // _TTOK
    return pl.pallas_call(
        _tr_body,
        grid=(grid,),
        in_specs=[pl.BlockSpec((E, _TTOK), lambda i: (0, i))],
        out_specs=pl.BlockSpec((_QT, 128), lambda i: (i, 0)),
        out_shape=jax.ShapeDtypeStruct((grid * _QT, 128), jnp.int32),
    )(embT)


def _mm_body(y_ref, wt_ref, b_ref, o_ref):
    o_ref[...] = jnp.dot(
        y_ref[...], wt_ref[...],
        preferred_element_type=jnp.float32,
        precision=lax.Precision.HIGHEST,
    ) + b_ref[...]


def _matmul(y, Wt, b2):
    B, E = y.shape
    N = Wt.shape[1]
    BB = 1024
    return pl.pallas_call(
        _mm_body,
        grid=(B // BB,),
        in_specs=[
            pl.BlockSpec((BB, E), lambda i: (i, 0)),
            pl.BlockSpec((E, N), lambda i: (0, 0)),
            pl.BlockSpec((1, N), lambda i: (0, 0)),
        ],
        out_specs=pl.BlockSpec((BB, N), lambda i: (i, 0)),
        out_shape=jax.ShapeDtypeStruct((B, N), jnp.float32),
    )(y, Wt, b2)


def kernel(input, embed, W, b):
    B, L = input.shape
    V, E = embed.shape
    v = input.astype(jnp.int32)
    r = (v & ~jnp.int32(_TTOK - 1)) + ((v & (_QT - 1)) << 2) \
        + ((v >> 10) & 3)
    ids2 = r.reshape(2 * B, L // 2)
    packed = _transpose_pack(embed.T)
    table2 = packed.reshape(4 * packed.shape[0], E // 2)
    y = _make_pool(B, L, E)(ids2, table2)
    perm = jnp.concatenate([
        jnp.arange(0, 32, 2), jnp.arange(1, 32, 2),
        jnp.arange(32, 64, 2), jnp.arange(33, 64, 2)])
    return _matmul(y, W.T[perm, :], b.reshape(1, -1))
